# Initial kernel scaffold; baseline (speedup 1.0000x reference)
import jax
import jax.numpy as jnp
from jax.experimental import pallas as pl


def kernel(A_indices, A_values, X, W, b):
    raise NotImplementedError("write your pallas kernel here")


# speedup vs baseline: 3.0229x; 3.0229x over previous
import functools

import jax
import jax.numpy as jnp
from jax import lax
from jax.experimental import pallas as pl
from jax.experimental.pallas import tpu as pltpu
from jax.experimental.pallas import tpu_sc as plsc

N = 10000
DIN = 128
DOUT = 128
E = 320000

NC = 2
NS = 16
NW = NC * NS
K = 128
NCH = 80
EPAD = NW * NCH * K
ACC_N = 10240
RPT = ACC_N // NS
LANES = 16
MBLK = 1000


def _linear_body(x_ref, wt_ref, b_ref, o_ref):
    o_ref[...] = (
        jnp.dot(x_ref[...], wt_ref[...], preferred_element_type=jnp.float32)
        + b_ref[...]
    )


def _linear(x, wt, b2):
    return pl.pallas_call(
        _linear_body,
        grid=(N // MBLK,),
        in_specs=[
            pl.BlockSpec((MBLK, DIN), lambda i: (i, 0)),
            pl.BlockSpec((DIN, DOUT), lambda i: (0, 0)),
            pl.BlockSpec((1, DOUT), lambda i: (0, 0)),
        ],
        out_specs=pl.BlockSpec((MBLK, DOUT), lambda i: (i, 0)),
        out_shape=jax.ShapeDtypeStruct((N, DOUT), jnp.float32),
    )(x, wt, b2)


def _combine_body(p_ref, o_ref):
    o_ref[...] = p_ref[0] + p_ref[1]


def _combine(partials):
    return pl.pallas_call(
        _combine_body,
        grid=(N // MBLK,),
        in_specs=[pl.BlockSpec((NC, MBLK, DOUT), lambda i: (0, i, 0))],
        out_specs=pl.BlockSpec((MBLK, DOUT), lambda i: (i, 0)),
        out_shape=jax.ShapeDtypeStruct((N, DOUT), jnp.float32),
    )(partials)


def _make_spmm():
    mesh = plsc.VectorSubcoreMesh(core_axis_name="c", subcore_axis_name="s")

    @functools.partial(
        pl.kernel,
        mesh=mesh,
        out_type=jax.ShapeDtypeStruct((NC, ACC_N, DOUT), jnp.float32),
        scratch_types=[
            pltpu.VMEM((NCH, K), jnp.int32),
            pltpu.VMEM((NCH, K), jnp.int32),
            pltpu.VMEM((NCH, K), jnp.float32),
            pltpu.VMEM((K, DOUT), jnp.float32),
            pltpu.VMEM_SHARED((ACC_N, DOUT), jnp.float32),
            pltpu.SemaphoreType.DMA,
        ],
    )
    def spmm(wx_hbm, cols_hbm, rows_hbm, vals_hbm, out_hbm,
             cols_v, rows_v, vals_v, gbuf, acc, sem):
        cid = lax.axis_index("c")
        sid = lax.axis_index("s")
        wid = cid * NS + sid

        pltpu.sync_copy(cols_hbm.at[wid], cols_v)
        pltpu.sync_copy(rows_hbm.at[wid], rows_v)
        pltpu.sync_copy(vals_hbm.at[wid], vals_v)

        def zrow(r, carry):
            for c in range(DOUT // LANES):
                gbuf[r, pl.ds(c * LANES, LANES)] = jnp.zeros(
                    (LANES,), jnp.float32)
            return carry
        lax.fori_loop(0, K, zrow, 0)
        base = sid * RPT
        for t in range(RPT // K):
            pltpu.sync_copy(gbuf, acc.at[pl.ds(base + t * K, K)])
        plsc.subcore_barrier()

        def chunk(j, carry):
            pltpu.async_copy(wx_hbm.at[cols_v.at[j]], gbuf, sem).wait()

            def rgroup(rg, c2):
                base_r = pl.multiple_of(rg * LANES, LANES)
                vrow = vals_v[j, pl.ds(base_r, LANES)]
                for r16 in range(LANES):
                    splat = vrow.at[jnp.full((LANES,), r16, jnp.int32)].get(
                        mode="promise_in_bounds")
                    rr = base_r + r16
                    for c in range(DOUT // LANES):
                        sl = pl.ds(c * LANES, LANES)
                        gbuf[rr, sl] = gbuf[rr, sl] * splat
                return c2
            lax.fori_loop(0, K // LANES, rgroup, 0)

            pltpu.sync_copy(gbuf, acc.at[rows_v.at[j]], add=True)
            return carry
        lax.fori_loop(0, NCH, chunk, 0)

        plsc.subcore_barrier()
        pltpu.sync_copy(acc.at[pl.ds(base, RPT)],
                        out_hbm.at[cid, pl.ds(base, RPT)])

    return spmm


_spmm = _make_spmm()


def kernel(A_indices, A_values, X, W, b):
    wx = _linear(X, W.T, b.reshape(1, DOUT))
    pad = EPAD - E
    rows = jnp.concatenate(
        [A_indices[0], jnp.zeros((pad,), jnp.int32)]).reshape(NW, NCH, K)
    cols = jnp.concatenate(
        [A_indices[1], jnp.zeros((pad,), jnp.int32)]).reshape(NW, NCH, K)
    vals = jnp.concatenate(
        [A_values, jnp.zeros((pad,), jnp.float32)]).reshape(NW, NCH, K)
    partials = _spmm(wx, cols, rows, vals)
    return _combine(partials)
